# baseline (speedup 1.0000x reference)
import jax
import jax.numpy as jnp
from jax.experimental import pallas as pl

PASS = 8
FLOW = 16
TRANS = FLOW - PASS
NBINS = 64
HID = 64
BLOCK = 1024


def _coupling_kernel(y_ref, W1_ref, b1_ref, W2_ref, b2_ref, W3_ref, b3_ref,
                     out_ref):
    y = y_ref[...]
    yA = y[:, :PASS]
    bs = yA.shape[0]

    h = jnp.maximum(
        jnp.dot(yA, W1_ref[...], preferred_element_type=jnp.float32)
        + b1_ref[...], 0.0)
    h = jnp.maximum(
        jnp.dot(h, W2_ref[...], preferred_element_type=jnp.float32)
        + b2_ref[...], 0.0)
    logits = (jnp.dot(h, W3_ref[...], preferred_element_type=jnp.float32)
              + b3_ref[...])
    Q = jnp.exp(logits)

    tri = (jax.lax.broadcasted_iota(jnp.int32, (NBINS, NBINS), 0)
           <= jax.lax.broadcasted_iota(jnp.int32, (NBINS, NBINS), 1)
           ).astype(jnp.float32)
    lane = jax.lax.broadcasted_iota(jnp.int32, (bs, NBINS), 1)

    xB_cols = []
    inv_slope_prod = jnp.ones((bs, 1), jnp.float32)
    for t in range(TRANS):
        Qt = Q[:, t * NBINS:(t + 1) * NBINS]
        Qsum = jnp.dot(Qt, tri, preferred_element_type=jnp.float32)
        Qnorm = Qsum[:, NBINS - 1:NBINS]
        Qsum = Qsum / Qnorm
        Qn = Qt * (NBINS / Qnorm)
        yB = y[:, PASS + t:PASS + t + 1]
        lt = Qsum < yB
        ybins = jnp.sum(lt.astype(jnp.int32), axis=1, keepdims=True)
        offset = jnp.max(jnp.where(lt, Qsum, 0.0), axis=1, keepdims=True)
        slope = jnp.sum(jnp.where(lane == ybins, Qn, 0.0), axis=1,
                        keepdims=True)
        xB_cols.append((yB - offset) / slope
                       + ybins.astype(jnp.float32) * (1.0 / NBINS))
        inv_slope_prod = inv_slope_prod * (1.0 / slope)

    jac = y[:, FLOW:FLOW + 1] * inv_slope_prod
    out_ref[...] = jnp.concatenate([yA] + xB_cols + [jac], axis=1)


def kernel(y, W1, b1, W2, b2, W3, b3):
    B = y.shape[0]
    grid = (B // BLOCK,)
    return pl.pallas_call(
        _coupling_kernel,
        grid=grid,
        in_specs=[
            pl.BlockSpec((BLOCK, FLOW + 1), lambda i: (i, 0)),
            pl.BlockSpec((PASS, HID), lambda i: (0, 0)),
            pl.BlockSpec((1, HID), lambda i: (0, 0)),
            pl.BlockSpec((HID, HID), lambda i: (0, 0)),
            pl.BlockSpec((1, HID), lambda i: (0, 0)),
            pl.BlockSpec((HID, TRANS * NBINS), lambda i: (0, 0)),
            pl.BlockSpec((1, TRANS * NBINS), lambda i: (0, 0)),
        ],
        out_specs=pl.BlockSpec((BLOCK, FLOW + 1), lambda i: (i, 0)),
        out_shape=jax.ShapeDtypeStruct((B, FLOW + 1), jnp.float32),
    )(y, W1, b1.reshape(1, HID), W2, b2.reshape(1, HID), W3,
      b3.reshape(1, TRANS * NBINS))


# speedup vs baseline: 7.1649x; 7.1649x over previous
import jax
import jax.numpy as jnp
from jax.experimental import pallas as pl

PASS = 8
FLOW = 16
TRANS = FLOW - PASS
NBINS = 64
HID = 64
BLOCK = 2048
BIG = 3.0e38


def _coupling_kernel(yT_ref, W1T_ref, b1_ref, W2T_ref, b2_ref, W3T_ref,
                     b3_ref, out_ref):
    yT = yT_ref[...]
    yAT = yT[:PASS, :]

    h = jnp.maximum(
        jnp.dot(W1T_ref[...], yAT, preferred_element_type=jnp.float32)
        + b1_ref[...], 0.0)
    h = jnp.maximum(
        jnp.dot(W2T_ref[...], h, preferred_element_type=jnp.float32)
        + b2_ref[...], 0.0)
    logits = (jnp.dot(W3T_ref[...], h, preferred_element_type=jnp.float32)
              + b3_ref[...])
    Q = jnp.exp(logits)

    tri = (jax.lax.broadcasted_iota(jnp.int32, (NBINS, NBINS), 0)
           >= jax.lax.broadcasted_iota(jnp.int32, (NBINS, NBINS), 1)
           ).astype(jnp.float32)

    rows = []
    inv_prod = yT[FLOW:FLOW + 1, :]
    for t in range(TRANS):
        Qt = Q[t * NBINS:(t + 1) * NBINS, :]
        Qs = jnp.dot(tri, Qt, preferred_element_type=jnp.float32)
        Qnorm = Qs[NBINS - 1:NBINS, :]
        u = yT[PASS + t:PASS + t + 1, :] * Qnorm
        Qs63 = Qs[:NBINS - 1, :]
        lt = Qs63 < u
        ybins = jnp.sum(lt.astype(jnp.float32), axis=0, keepdims=True)
        offset = jnp.max(jnp.where(lt, Qs63, 0.0), axis=0, keepdims=True)
        nxt = jnp.min(jnp.where(lt, BIG, Qs63), axis=0, keepdims=True)
        nxt = jnp.minimum(nxt, Qnorm)
        slope64 = (nxt - offset) * float(NBINS)
        rows.append((u - offset) / slope64 + ybins * (1.0 / NBINS))
        inv_prod = inv_prod * (Qnorm / slope64)

    out_ref[...] = jnp.concatenate(rows + [inv_prod], axis=0)


def kernel(y, W1, b1, W2, b2, W3, b3):
    B = y.shape[0]
    grid = (B // BLOCK,)
    o9 = pl.pallas_call(
        _coupling_kernel,
        grid=grid,
        in_specs=[
            pl.BlockSpec((FLOW + 1, BLOCK), lambda i: (0, i)),
            pl.BlockSpec((HID, PASS), lambda i: (0, 0)),
            pl.BlockSpec((HID, 1), lambda i: (0, 0)),
            pl.BlockSpec((HID, HID), lambda i: (0, 0)),
            pl.BlockSpec((HID, 1), lambda i: (0, 0)),
            pl.BlockSpec((TRANS * NBINS, HID), lambda i: (0, 0)),
            pl.BlockSpec((TRANS * NBINS, 1), lambda i: (0, 0)),
        ],
        out_specs=pl.BlockSpec((TRANS + 1, BLOCK), lambda i: (0, i)),
        out_shape=jax.ShapeDtypeStruct((TRANS + 1, B), jnp.float32),
    )(y.T, W1.T, b1[:, None], W2.T, b2[:, None], W3.T, b3[:, None])
    return jnp.concatenate([y[:, :PASS], o9.T], axis=1)
